# Initial kernel scaffold; baseline (speedup 1.0000x reference)
import jax
import jax.numpy as jnp
from jax.experimental import pallas as pl


def kernel(noise_key, x, W_g, W_noise):
    raise NotImplementedError("write your pallas kernel here")


# speedup vs baseline: 1.3684x; 1.3684x over previous
import functools

import jax
import jax.numpy as jnp
from jax.experimental import pallas as pl
from jax.experimental.pallas import tpu as pltpu

TOKENS = 32768
D_MODEL = 4096
N_MODELS = 64
TOPK = 8
BLOCK_T = 512


def _router_body(x_ref, w_ref, nz_ref, o_ref):
    acc = jnp.dot(x_ref[:], w_ref[:], preferred_element_type=jnp.float32)
    hg = acc[:, :N_MODELS]
    sp = acc[:, N_MODELS:]
    h = hg + nz_ref[:] * jnp.logaddexp(sp, 0.0)

    col = jax.lax.broadcasted_iota(jnp.int32, h.shape, 1)
    neg_inf = jnp.float32(-jnp.inf)
    hm = h
    row_max = jnp.max(hm, axis=1, keepdims=True)
    m = row_max
    for _ in range(TOPK - 1):
        first = jnp.min(jnp.where(hm == m, col, N_MODELS), axis=1, keepdims=True)
        hm = jnp.where(col == first, neg_inf, hm)
        m = jnp.max(hm, axis=1, keepdims=True)
    kth = m

    e = jnp.where(h >= kth, jnp.exp(h - row_max), 0.0)
    o_ref[:] = e / jnp.sum(e, axis=1, keepdims=True)


@functools.partial(jax.jit, static_argnames=())
def _run(x, w_cat, noise):
    grid = (TOKENS // BLOCK_T,)
    return pl.pallas_call(
        _router_body,
        grid=grid,
        in_specs=[
            pl.BlockSpec((BLOCK_T, D_MODEL), lambda i: (i, 0)),
            pl.BlockSpec((D_MODEL, 2 * N_MODELS), lambda i: (0, 0)),
            pl.BlockSpec((BLOCK_T, N_MODELS), lambda i: (i, 0)),
        ],
        out_specs=pl.BlockSpec((BLOCK_T, N_MODELS), lambda i: (i, 0)),
        out_shape=jax.ShapeDtypeStruct((TOKENS, N_MODELS), jnp.float32),
        compiler_params=pltpu.CompilerParams(
            dimension_semantics=("arbitrary",),
        ),
    )(x, w_cat, noise)


def kernel(noise_key, x, W_g, W_noise):
    x2 = x if x.ndim == 2 else x.reshape((x.shape[0], -1))
    noise = jax.random.normal(noise_key, shape=(x2.shape[0], N_MODELS))
    w_cat = jnp.concatenate([W_g, W_noise], axis=1)
    return _run(x2, w_cat, noise)


# speedup vs baseline: 1.3734x; 1.0036x over previous
import functools

import jax
import jax.numpy as jnp
from jax.experimental import pallas as pl
from jax.experimental.pallas import tpu as pltpu

TOKENS = 32768
D_MODEL = 4096
N_MODELS = 64
TOPK = 8
BLOCK_T = 512


def _router_body(x_ref, w_ref, nz_ref, o_ref):
    acc = jnp.dot(
        x_ref[:].astype(jnp.bfloat16),
        w_ref[:],
        preferred_element_type=jnp.float32,
    )
    hg = acc[:, :N_MODELS]
    sp = acc[:, N_MODELS:]
    h = hg + nz_ref[:] * jnp.logaddexp(sp, 0.0)

    col = jax.lax.broadcasted_iota(jnp.int32, h.shape, 1)
    neg_inf = jnp.float32(-jnp.inf)
    hm = h
    row_max = jnp.max(hm, axis=1, keepdims=True)
    m = row_max
    for _ in range(TOPK - 1):
        first = jnp.min(jnp.where(hm == m, col, N_MODELS), axis=1, keepdims=True)
        hm = jnp.where(col == first, neg_inf, hm)
        m = jnp.max(hm, axis=1, keepdims=True)
    kth = m

    e = jnp.where(h >= kth, jnp.exp(h - row_max), 0.0)
    o_ref[:] = e / jnp.sum(e, axis=1, keepdims=True)


@functools.partial(jax.jit, static_argnames=())
def _run(x, w_cat, noise):
    grid = (TOKENS // BLOCK_T,)
    return pl.pallas_call(
        _router_body,
        grid=grid,
        in_specs=[
            pl.BlockSpec((BLOCK_T, D_MODEL), lambda i: (i, 0)),
            pl.BlockSpec((D_MODEL, 2 * N_MODELS), lambda i: (0, 0)),
            pl.BlockSpec((BLOCK_T, N_MODELS), lambda i: (i, 0)),
        ],
        out_specs=pl.BlockSpec((BLOCK_T, N_MODELS), lambda i: (i, 0)),
        out_shape=jax.ShapeDtypeStruct((TOKENS, N_MODELS), jnp.float32),
        compiler_params=pltpu.CompilerParams(
            dimension_semantics=("arbitrary",),
        ),
    )(x, w_cat, noise)


def kernel(noise_key, x, W_g, W_noise):
    x2 = x if x.ndim == 2 else x.reshape((x.shape[0], -1))
    noise = jax.random.normal(noise_key, shape=(x2.shape[0], N_MODELS))
    w_cat = jnp.concatenate([W_g, W_noise], axis=1).astype(jnp.bfloat16)
    return _run(x2, w_cat, noise)


# speedup vs baseline: 1.4436x; 1.0511x over previous
import functools

import jax
import jax.numpy as jnp
from jax.experimental import pallas as pl
from jax.experimental.pallas import tpu as pltpu

TOKENS = 32768
D_MODEL = 4096
N_MODELS = 64
TOPK = 8
BLOCK_T = 1024


def _router_body(x_ref, w_ref, nz_ref, o_ref):
    acc = jnp.dot(
        x_ref[:].astype(jnp.bfloat16),
        w_ref[:],
        preferred_element_type=jnp.float32,
    )
    hg = acc[:, :N_MODELS]
    sp = acc[:, N_MODELS:]
    h = hg + nz_ref[:] * jnp.logaddexp(sp, 0.0)

    col = jax.lax.broadcasted_iota(jnp.int32, h.shape, 1)
    neg_inf = jnp.float32(-jnp.inf)
    hm = h
    row_max = jnp.max(hm, axis=1, keepdims=True)
    m = row_max
    for _ in range(TOPK - 1):
        first = jnp.min(jnp.where(hm == m, col, N_MODELS), axis=1, keepdims=True)
        hm = jnp.where(col == first, neg_inf, hm)
        m = jnp.max(hm, axis=1, keepdims=True)
    kth = m

    e = jnp.where(h >= kth, jnp.exp(h - row_max), 0.0)
    o_ref[:] = e / jnp.sum(e, axis=1, keepdims=True)


def _dense_only_body(x_ref, w_ref, nz_ref, o_ref):
    acc = jnp.dot(
        x_ref[:].astype(jnp.bfloat16),
        w_ref[:],
        preferred_element_type=jnp.float32,
    )
    hg = acc[:, :N_MODELS]
    sp = acc[:, N_MODELS:]
    o_ref[:] = hg + nz_ref[:] * jnp.logaddexp(sp, 0.0)


@functools.partial(jax.jit, static_argnames=())
def _run(x, w_cat, noise):
    grid = (TOKENS // BLOCK_T,)
    return pl.pallas_call(
        _router_body,
        grid=grid,
        in_specs=[
            pl.BlockSpec((BLOCK_T, D_MODEL), lambda i: (i, 0)),
            pl.BlockSpec((D_MODEL, 2 * N_MODELS), lambda i: (0, 0)),
            pl.BlockSpec((BLOCK_T, N_MODELS), lambda i: (i, 0)),
        ],
        out_specs=pl.BlockSpec((BLOCK_T, N_MODELS), lambda i: (i, 0)),
        out_shape=jax.ShapeDtypeStruct((TOKENS, N_MODELS), jnp.float32),
        compiler_params=pltpu.CompilerParams(
            dimension_semantics=("arbitrary",),
        ),
    )(x, w_cat, noise)


def kernel(noise_key, x, W_g, W_noise):
    x2 = x if x.ndim == 2 else x.reshape((x.shape[0], -1))
    noise = jax.random.normal(noise_key, shape=(x2.shape[0], N_MODELS))
    w_cat = jnp.concatenate([W_g, W_noise], axis=1).astype(jnp.bfloat16)
    return _run(x2, w_cat, noise)
